# baseline (speedup 1.0000x reference)
import functools

import jax
import jax.numpy as jnp
from jax import lax
from jax.experimental import pallas as pl
from jax.experimental.pallas import tpu as pltpu
from jax.experimental.pallas import tpu_sc as plsc

_N = 16384
_G = 1001
_NC = 1
_NS = 16
_NW = _NC * _NS
_CHUNK = _N // _NW
_L = 16
_ROW = 128
_NROW = _CHUNK // _ROW


def _sc_body(t_hbm, gamma_hbm, out_hbm, t_v, idx_v, o_v, sem_t, sem_g):
    wid = lax.axis_index("s") * _NC + lax.axis_index("c")
    base = wid * _CHUNK
    pltpu.async_copy(t_hbm.at[pl.ds(base, _CHUNK)], t_v, sem_t).wait()
    for r in range(_NROW):
        for k in range(_ROW // _L):
            x = t_v[pl.ds(r * _ROW + k * _L, _L)] * 1000.0
            i0 = x.astype(jnp.int32)
            frac = x - i0.astype(jnp.float32)
            up = (frac > 0.5) | ((frac == 0.5) & ((i0 & 1) == 1))
            idx_v[r, pl.ds(k * _L, _L)] = jnp.where(up, i0 + 1, i0)
    copies = [
        pltpu.async_copy(
            gamma_hbm.at[idx_v.at[r]], o_v.at[pl.ds(r * _ROW, _ROW)], sem_g
        )
        for r in range(_NROW)
    ]
    for cp in copies:
        cp.wait()
    pltpu.sync_copy(o_v, out_hbm.at[pl.ds(base, _CHUNK)])


@jax.jit
def kernel(t, gamma):
    mesh = plsc.VectorSubcoreMesh(
        core_axis_name="c", subcore_axis_name="s", num_cores=_NC
    )
    run = functools.partial(
        pl.kernel,
        out_type=jax.ShapeDtypeStruct((_N,), jnp.float32),
        mesh=mesh,
        scratch_types=[
            pltpu.VMEM((_CHUNK,), jnp.float32),
            pltpu.VMEM((_NROW, _ROW), jnp.int32),
            pltpu.VMEM((_CHUNK,), jnp.float32),
            pltpu.SemaphoreType.DMA,
            pltpu.SemaphoreType.DMA,
        ],
        compiler_params=pltpu.CompilerParams(
            needs_layout_passes=False, skip_device_barrier=True
        ),
    )(_sc_body)
    out = run(t.reshape(_N), gamma)
    return out.reshape(t.shape)


# speedup vs baseline: 1.4915x; 1.4915x over previous
import functools

import jax
import jax.numpy as jnp
from jax import lax
from jax.experimental import pallas as pl
from jax.experimental.pallas import tpu as pltpu
from jax.experimental.pallas import tpu_sc as plsc

_N = 16384
_G = 1001
_NC = 1
_NS = 16
_NW = _NC * _NS
_CHUNK = _N // _NW
_L = 16


def _sc_body(t_hbm, gamma_hbm, out_hbm, t_v, g_v, o_v, sem_g, sem_t):
    wid = lax.axis_index("s") * _NC + lax.axis_index("c")
    base = wid * _CHUNK
    cp_g = pltpu.async_copy(gamma_hbm, g_v, sem_g)
    cp_t = pltpu.async_copy(t_hbm.at[pl.ds(base, _CHUNK)], t_v, sem_t)
    cp_g.wait()
    cp_t.wait()
    for off in range(0, _CHUNK, _L):
        x = t_v[pl.ds(off, _L)] * 1000.0
        i0 = x.astype(jnp.int32)
        frac = x - i0.astype(jnp.float32)
        up = (frac > 0.5) | ((frac == 0.5) & ((i0 & 1) == 1))
        idx = jnp.where(up, i0 + 1, i0)
        o_v[pl.ds(off, _L)] = plsc.load_gather(g_v, [idx])
    pltpu.sync_copy(o_v, out_hbm.at[pl.ds(base, _CHUNK)])


@jax.jit
def kernel(t, gamma):
    mesh = plsc.VectorSubcoreMesh(
        core_axis_name="c", subcore_axis_name="s", num_cores=_NC
    )
    run = functools.partial(
        pl.kernel,
        out_type=jax.ShapeDtypeStruct((_N,), jnp.float32),
        mesh=mesh,
        scratch_types=[
            pltpu.VMEM((_CHUNK,), jnp.float32),
            pltpu.VMEM((_G,), jnp.float32),
            pltpu.VMEM((_CHUNK,), jnp.float32),
            pltpu.SemaphoreType.DMA,
            pltpu.SemaphoreType.DMA,
        ],
        compiler_params=pltpu.CompilerParams(
            needs_layout_passes=False, skip_device_barrier=True
        ),
    )(_sc_body)
    out = run(t.reshape(_N), gamma)
    return out.reshape(t.shape)
